# baseline (speedup 1.0000x reference)
import functools

import jax
import jax.numpy as jnp
from jax import lax
from jax.experimental import pallas as pl
from jax.experimental.pallas import tpu as pltpu
from jax.experimental.pallas import tpu_sc as plsc

N = 10000
E = 320000
D = 128
NC, NS = 2, 16
NW = NC * NS
CHUNK = 128

NP_ = 10240
RPT = NP_ // NW
SPLIT = 5120
ACC_R = 5120
RPT_A = ACC_R // NW
A_CH = 160
A_PAD = NW * A_CH * CHUNK
E_CH = 80
E_PAD = NW * E_CH * CHUNK

_mesh = plsc.VectorSubcoreMesh(
    core_axis_name="c", subcore_axis_name="s", num_cores=NC, num_subcores=NS)


_DEG_K = 8


@functools.partial(
    pl.kernel,
    out_type=jax.ShapeDtypeStruct((NC * NP_,), jnp.float32),
    mesh=_mesh,
    scratch_types=[
        pltpu.VMEM((A_CH, CHUNK), jnp.int32),
        pltpu.VMEM((CHUNK,), jnp.float32),
        pltpu.VMEM((RPT,), jnp.float32),
        pltpu.VMEM_SHARED((NP_,), jnp.float32),
        pltpu.SemaphoreType.DMA,
    ],
)
def _deg_kernel(dst_hbm, ones_hbm, zeros_hbm, degp_hbm, idx_v, ones_v, db_v,
                acc_sh, dsem):
    cid = lax.axis_index("c")
    sid = lax.axis_index("s")
    wid = sid * NC + cid
    r0 = sid * RPT
    pltpu.sync_copy(ones_hbm, ones_v)
    pltpu.sync_copy(zeros_hbm.at[pl.ds(r0, RPT)], db_v)
    pltpu.sync_copy(db_v, acc_sh.at[pl.ds(r0, RPT)])
    pltpu.sync_copy(dst_hbm.at[wid], idx_v)
    plsc.subcore_barrier()

    def body(j, carry):
        pltpu.async_copy(ones_v, acc_sh.at[idx_v.at[j]], dsem, add=True)

        @pl.when(j >= _DEG_K)
        def _():
            pltpu.make_async_copy(ones_v, acc_sh.at[idx_v.at[j - _DEG_K]],
                                  dsem).wait()
        return carry

    lax.fori_loop(0, A_CH, body, 0)

    def drain(j, carry):
        pltpu.make_async_copy(ones_v, acc_sh.at[idx_v.at[j]], dsem).wait()
        return carry

    lax.fori_loop(A_CH - _DEG_K, A_CH, drain, 0)
    plsc.subcore_barrier()
    pltpu.sync_copy(acc_sh.at[pl.ds(r0, RPT)], db_v)
    pltpu.sync_copy(db_v, degp_hbm.at[pl.ds(cid * NP_ + r0, RPT)])


AGG_C = 320
A_G = A_PAD // (NW * AGG_C)


@functools.partial(
    pl.kernel,
    out_type=jax.ShapeDtypeStruct((NC, ACC_R, D), jnp.float32),
    mesh=_mesh,
    scratch_types=[
        pltpu.VMEM((A_G * AGG_C,), jnp.int32),
        pltpu.VMEM((A_G * AGG_C,), jnp.int32),
        pltpu.VMEM((AGG_C, D), jnp.float32),
        pltpu.VMEM_SHARED((ACC_R, D), jnp.float32),
    ],
)
def _agg_kernel(z_hbm, src_hbm, dst_hbm, zeros_hbm, aggp_hbm,
                sidx_v, didx_v, rows_v, acc_sh):
    cid = lax.axis_index("c")
    sid = lax.axis_index("s")
    wid = sid * NC + cid
    r0 = sid * RPT_A
    pltpu.sync_copy(src_hbm.at[wid], sidx_v)
    pltpu.sync_copy(dst_hbm.at[wid], didx_v)
    pltpu.sync_copy(zeros_hbm.at[pl.ds(r0, RPT_A)],
                    rows_v.at[pl.ds(0, RPT_A)])
    pltpu.sync_copy(rows_v.at[pl.ds(0, RPT_A)], acc_sh.at[pl.ds(r0, RPT_A)])
    plsc.subcore_barrier()

    def body(g, carry):
        sl = pl.ds(g * AGG_C, AGG_C)
        pltpu.sync_copy(z_hbm.at[sidx_v.at[sl]], rows_v)
        pltpu.sync_copy(rows_v, acc_sh.at[didx_v.at[sl]], add=True)
        return carry

    lax.fori_loop(0, A_G, body, 0)
    plsc.subcore_barrier()
    pltpu.sync_copy(acc_sh.at[pl.ds(r0, RPT_A)], rows_v.at[pl.ds(0, RPT_A)])
    pltpu.sync_copy(rows_v.at[pl.ds(0, RPT_A)],
                    aggp_hbm.at[cid, pl.ds(r0, RPT_A)])


_QR = 3


@functools.partial(
    pl.kernel,
    out_type=jax.ShapeDtypeStruct((E_PAD, D), jnp.float32),
    mesh=_mesh,
    scratch_types=[
        pltpu.VMEM((E_CH, CHUNK), jnp.int32),
        pltpu.VMEM((E_CH, CHUNK), jnp.int32),
        pltpu.VMEM((_QR, 2 * CHUNK, D), jnp.float32),
        pltpu.SemaphoreType.DMA((_QR,)),
        pltpu.SemaphoreType.DMA((_QR,)),
        pltpu.SemaphoreType.DMA((_QR,)),
    ],
)
def _q_kernel(p_hbm, pneg_hbm, row_hbm, col_hbm, q_hbm,
              ridx_v, cidx_v, buf, ga, gb, wsem):
    cid = lax.axis_index("c")
    sid = lax.axis_index("s")
    wid = sid * NC + cid
    base = wid * E_CH * CHUNK
    pltpu.sync_copy(row_hbm.at[wid], ridx_v)
    pltpu.sync_copy(col_hbm.at[wid], cidx_v)

    def issue(j, s):
        pltpu.async_copy(p_hbm.at[ridx_v.at[j]],
                         buf.at[s, pl.ds(0, CHUNK)], ga.at[s])
        pltpu.async_copy(pneg_hbm.at[cidx_v.at[j]],
                         buf.at[s, pl.ds(CHUNK, CHUNK)], gb.at[s])

    for b in range(_QR - 1):
        issue(b, b)

    def body(j, carry):
        s = lax.rem(j, _QR)
        pltpu.make_async_copy(p_hbm.at[ridx_v.at[j]],
                              buf.at[s, pl.ds(0, CHUNK)], ga.at[s]).wait()
        pltpu.make_async_copy(pneg_hbm.at[cidx_v.at[j]],
                              buf.at[s, pl.ds(CHUNK, CHUNK)], gb.at[s]).wait()

        def row_body(r, c2):
            for c in range(D // 16):
                sl = pl.ds(c * 16, 16)
                plsc.addupdate(buf.at[s, r, sl], buf[s, CHUNK + r, sl])
            return c2

        lax.fori_loop(0, CHUNK, row_body, 0)
        pltpu.async_copy(buf.at[s, pl.ds(0, CHUNK)],
                         q_hbm.at[pl.ds(base + j * CHUNK, CHUNK)], wsem.at[s])
        s2 = lax.rem(j + _QR - 1, _QR)

        @pl.when(jnp.logical_and(j >= 1, j + _QR - 1 < E_CH))
        def _():
            pltpu.make_async_copy(
                buf.at[s2, pl.ds(0, CHUNK)],
                q_hbm.at[pl.ds(base + (j - 1) * CHUNK, CHUNK)],
                wsem.at[s2]).wait()

        @pl.when(j + _QR - 1 < E_CH)
        def _():
            issue(j + _QR - 1, s2)
        return carry

    lax.fori_loop(0, E_CH, body, 0)

    def drain(j, carry):
        s = lax.rem(j, _QR)
        pltpu.make_async_copy(buf.at[s, pl.ds(0, CHUNK)],
                              q_hbm.at[pl.ds(base + j * CHUNK, CHUNK)],
                              wsem.at[s]).wait()
        return carry

    lax.fori_loop(E_CH - _QR, E_CH, drain, 0)


BLK = 1024
EBLK = 2048


def _k1_body(x_ref, w1_ref, degp_ref, z1_ref, dinv_ref):
    deg = degp_ref[:, 0:1] + degp_ref[:, 1:2] + 1.0
    dinv = lax.rsqrt(deg)
    y = jnp.dot(x_ref[...], w1_ref[...], preferred_element_type=jnp.float32)
    z1_ref[...] = y * dinv
    dinv_ref[...] = dinv


def _k2_body(agg_ref, z1_ref, dinv_ref, b1_ref, w2_ref, z2_ref):
    agg = agg_ref[0] + agg_ref[1]
    dinv = dinv_ref[...]
    f1 = jnp.maximum((agg + z1_ref[...]) * dinv + b1_ref[...], 0.0)
    z2_ref[...] = jnp.dot(f1, w2_ref[...],
                          preferred_element_type=jnp.float32) * dinv


def _k3_body(agg_ref, z2_ref, dinv_ref, b2_ref, m1_ref, p_ref, pneg_ref):
    agg = agg_ref[0] + agg_ref[1]
    dinv = dinv_ref[...]
    f2 = jnp.maximum((agg + z2_ref[...]) * dinv + b2_ref[...], 0.0)
    p = jnp.dot(f2, m1_ref[...], preferred_element_type=jnp.float32)
    p_ref[...] = p
    pneg_ref[...] = -p


def _k4_body(q_ref, c1_ref, m2_ref, c2_ref, m3_ref, c3_ref, o_ref):
    h1 = jnp.maximum(q_ref[...] + c1_ref[...], 0.0)
    h2 = jnp.maximum(
        jnp.dot(h1, m2_ref[...], preferred_element_type=jnp.float32)
        + c2_ref[...], 0.0)
    o = jnp.dot(h2, m3_ref[...], preferred_element_type=jnp.float32)
    o_ref[...] = jax.nn.sigmoid(o + c3_ref[...])


def _row_spec(i):
    return (i, 0)


def _rep_spec(i):
    return (0, 0)


def _node_spec(i):
    return (0, i, 0)


def _aggregate(z, srcA_r, srcB_r, dstm_r, zerosA):
    aggA = _agg_kernel(z, srcA_r, dstm_r, zerosA)
    aggB = _agg_kernel(z, srcB_r, dstm_r, zerosA)
    return jnp.concatenate([aggA, aggB], axis=1)


def kernel(x, edge_index, W1, b1, W2, b2, M1, c1, M2, c2, M3, c3):
    f32 = jnp.float32
    ei = edge_index.astype(jnp.int32)
    row, col = ei[:, 0], ei[:, 1]
    fill = jnp.full((A_PAD - 2 * E,), NP_ - 1, jnp.int32)
    src_u = jnp.concatenate([row, col, fill])
    dst_u = jnp.concatenate([col, row, fill])
    inA = dst_u < SPLIT
    dstm_r = jnp.where(inA, dst_u, dst_u - SPLIT).reshape(NW, A_G * AGG_C)
    srcA_r = jnp.where(inA, src_u, NP_ - 1).reshape(NW, A_G * AGG_C)
    srcB_r = jnp.where(inA, NP_ - 1, src_u).reshape(NW, A_G * AGG_C)
    efill = jnp.zeros((E_PAD - E,), jnp.int32)
    row_r = jnp.concatenate([row, efill]).reshape(NW, E_CH, CHUNK)
    col_r = jnp.concatenate([col, efill]).reshape(NW, E_CH, CHUNK)

    zerosA = jnp.zeros((ACC_R, D), f32)
    zeros1d = jnp.zeros((NP_,), f32)
    ones1d = jnp.ones((CHUNK,), f32)
    x_pad = jnp.concatenate([x, jnp.zeros((NP_ - N, D), f32)], axis=0)

    degp = _deg_kernel(dst_r := dst_u.reshape(NW, A_CH, CHUNK), ones1d,
                       zeros1d)
    degp_t = degp.reshape(NC, NP_).T

    grid_n = NP_ // BLK
    z1, dinv = pl.pallas_call(
        _k1_body,
        grid=(grid_n,),
        in_specs=[
            pl.BlockSpec((BLK, D), _row_spec),
            pl.BlockSpec((D, D), _rep_spec),
            pl.BlockSpec((BLK, NC), _row_spec),
        ],
        out_specs=[
            pl.BlockSpec((BLK, D), _row_spec),
            pl.BlockSpec((BLK, 1), _row_spec),
        ],
        out_shape=[
            jax.ShapeDtypeStruct((NP_, D), f32),
            jax.ShapeDtypeStruct((NP_, 1), f32),
        ],
    )(x_pad, W1, degp_t)

    agg1 = _aggregate(z1, srcA_r, srcB_r, dstm_r, zerosA)

    z2 = pl.pallas_call(
        _k2_body,
        grid=(grid_n,),
        in_specs=[
            pl.BlockSpec((NC, BLK, D), _node_spec),
            pl.BlockSpec((BLK, D), _row_spec),
            pl.BlockSpec((BLK, 1), _row_spec),
            pl.BlockSpec((1, D), _rep_spec),
            pl.BlockSpec((D, D), _rep_spec),
        ],
        out_specs=pl.BlockSpec((BLK, D), _row_spec),
        out_shape=jax.ShapeDtypeStruct((NP_, D), f32),
    )(agg1, z1, dinv, b1.reshape(1, D), W2)

    agg2 = _aggregate(z2, srcA_r, srcB_r, dstm_r, zerosA)

    p, pneg = pl.pallas_call(
        _k3_body,
        grid=(grid_n,),
        in_specs=[
            pl.BlockSpec((NC, BLK, D), _node_spec),
            pl.BlockSpec((BLK, D), _row_spec),
            pl.BlockSpec((BLK, 1), _row_spec),
            pl.BlockSpec((1, D), _rep_spec),
            pl.BlockSpec((D, D), _rep_spec),
        ],
        out_specs=[
            pl.BlockSpec((BLK, D), _row_spec),
            pl.BlockSpec((BLK, D), _row_spec),
        ],
        out_shape=[
            jax.ShapeDtypeStruct((NP_, D), f32),
            jax.ShapeDtypeStruct((NP_, D), f32),
        ],
    )(agg2, z2, dinv, b2.reshape(1, D), M1)

    q = _q_kernel(p, pneg, row_r, col_r)

    m3p = jnp.concatenate([M3, jnp.zeros((D, 7), f32)], axis=1)
    c3t = jnp.broadcast_to(c3.reshape(1, 1), (1, 8))
    out = pl.pallas_call(
        _k4_body,
        grid=(E_PAD // EBLK,),
        in_specs=[
            pl.BlockSpec((EBLK, D), _row_spec),
            pl.BlockSpec((1, D), _rep_spec),
            pl.BlockSpec((D, D), _rep_spec),
            pl.BlockSpec((1, D), _rep_spec),
            pl.BlockSpec((D, 8), _rep_spec),
            pl.BlockSpec((1, 8), _rep_spec),
        ],
        out_specs=pl.BlockSpec((EBLK, 8), _row_spec),
        out_shape=jax.ShapeDtypeStruct((E_PAD, 8), f32),
    )(q, c1.reshape(1, D), M2, c2.reshape(1, D), m3p, c3t)

    return out[:E, :1]


# speedup vs baseline: 10.3214x; 10.3214x over previous
import functools

import jax
import jax.numpy as jnp
from jax import lax
from jax.experimental import pallas as pl
from jax.experimental.pallas import tpu as pltpu
from jax.experimental.pallas import tpu_sc as plsc

N = 10000
E = 320000
D = 128
NC, NS = 2, 16
NW = NC * NS
CHUNK = 128

NP_ = 10240
RPT = NP_ // NW
SPLIT = 5120
ACC_R = 5376
RPT_A = ACC_R // NW
A_CH = 160
A_PAD = NW * A_CH * CHUNK
E_CH = 80
E_PAD = NW * E_CH * CHUNK

_mesh = plsc.VectorSubcoreMesh(
    core_axis_name="c", subcore_axis_name="s", num_cores=NC, num_subcores=NS)


_DEG_K = 8


@functools.partial(
    pl.kernel,
    out_type=jax.ShapeDtypeStruct((NC * NP_,), jnp.float32),
    mesh=_mesh,
    scratch_types=[
        pltpu.VMEM((A_CH, CHUNK), jnp.int32),
        pltpu.VMEM((CHUNK,), jnp.float32),
        pltpu.VMEM((RPT,), jnp.float32),
        pltpu.VMEM_SHARED((NP_,), jnp.float32),
        pltpu.SemaphoreType.DMA,
    ],
)
def _deg_kernel(dst_hbm, ones_hbm, zeros_hbm, degp_hbm, idx_v, ones_v, db_v,
                acc_sh, dsem):
    cid = lax.axis_index("c")
    sid = lax.axis_index("s")
    wid = sid * NC + cid
    r0 = sid * RPT
    pltpu.sync_copy(ones_hbm, ones_v)
    pltpu.sync_copy(zeros_hbm.at[pl.ds(r0, RPT)], db_v)
    pltpu.sync_copy(db_v, acc_sh.at[pl.ds(r0, RPT)])
    pltpu.sync_copy(dst_hbm.at[wid], idx_v)
    plsc.subcore_barrier()

    def body(j, carry):
        pltpu.async_copy(ones_v, acc_sh.at[idx_v.at[j]], dsem, add=True)

        @pl.when(j >= _DEG_K)
        def _():
            pltpu.make_async_copy(ones_v, acc_sh.at[idx_v.at[j - _DEG_K]],
                                  dsem).wait()
        return carry

    lax.fori_loop(0, A_CH, body, 0)

    def drain(j, carry):
        pltpu.make_async_copy(ones_v, acc_sh.at[idx_v.at[j]], dsem).wait()
        return carry

    lax.fori_loop(A_CH - _DEG_K, A_CH, drain, 0)
    plsc.subcore_barrier()
    pltpu.sync_copy(acc_sh.at[pl.ds(r0, RPT)], db_v)
    pltpu.sync_copy(db_v, degp_hbm.at[pl.ds(cid * NP_ + r0, RPT)])


AGG_C = 320
A_G = A_PAD // (NW * AGG_C)


@functools.partial(
    pl.kernel,
    out_type=jax.ShapeDtypeStruct((NC, ACC_R, D), jnp.float32),
    mesh=_mesh,
    scratch_types=[
        pltpu.VMEM((A_G * AGG_C,), jnp.int32),
        pltpu.VMEM((A_G * AGG_C,), jnp.int32),
        pltpu.VMEM((AGG_C, D), jnp.float32),
        pltpu.VMEM_SHARED((ACC_R, D), jnp.float32),
    ],
)
def _agg_kernel(z_hbm, src_hbm, dst_hbm, zeros_hbm, aggp_hbm,
                sidx_v, didx_v, rows_v, acc_sh):
    cid = lax.axis_index("c")
    sid = lax.axis_index("s")
    wid = sid * NC + cid
    r0 = sid * RPT_A
    pltpu.sync_copy(src_hbm.at[wid], sidx_v)
    pltpu.sync_copy(dst_hbm.at[wid], didx_v)
    pltpu.sync_copy(zeros_hbm.at[pl.ds(r0, RPT_A)],
                    rows_v.at[pl.ds(0, RPT_A)])
    pltpu.sync_copy(rows_v.at[pl.ds(0, RPT_A)], acc_sh.at[pl.ds(r0, RPT_A)])
    plsc.subcore_barrier()

    def body(g, carry):
        sl = pl.ds(g * AGG_C, AGG_C)
        pltpu.sync_copy(z_hbm.at[sidx_v.at[sl]], rows_v)
        pltpu.sync_copy(rows_v, acc_sh.at[didx_v.at[sl]], add=True)
        return carry

    lax.fori_loop(0, A_G, body, 0)
    plsc.subcore_barrier()
    pltpu.sync_copy(acc_sh.at[pl.ds(r0, RPT_A)], rows_v.at[pl.ds(0, RPT_A)])
    pltpu.sync_copy(rows_v.at[pl.ds(0, RPT_A)],
                    aggp_hbm.at[cid, pl.ds(r0, RPT_A)])


_QR = 3


@functools.partial(
    pl.kernel,
    out_type=jax.ShapeDtypeStruct((E_PAD, D), jnp.float32),
    mesh=_mesh,
    scratch_types=[
        pltpu.VMEM((E_CH, CHUNK), jnp.int32),
        pltpu.VMEM((E_CH, CHUNK), jnp.int32),
        pltpu.VMEM((_QR, 2 * CHUNK, D), jnp.float32),
        pltpu.SemaphoreType.DMA((_QR,)),
        pltpu.SemaphoreType.DMA((_QR,)),
        pltpu.SemaphoreType.DMA((_QR,)),
    ],
)
def _q_kernel(p_hbm, pneg_hbm, row_hbm, col_hbm, q_hbm,
              ridx_v, cidx_v, buf, ga, gb, wsem):
    cid = lax.axis_index("c")
    sid = lax.axis_index("s")
    wid = sid * NC + cid
    base = wid * E_CH * CHUNK
    pltpu.sync_copy(row_hbm.at[wid], ridx_v)
    pltpu.sync_copy(col_hbm.at[wid], cidx_v)

    def issue(j, s):
        pltpu.async_copy(p_hbm.at[ridx_v.at[j]],
                         buf.at[s, pl.ds(0, CHUNK)], ga.at[s])
        pltpu.async_copy(pneg_hbm.at[cidx_v.at[j]],
                         buf.at[s, pl.ds(CHUNK, CHUNK)], gb.at[s])

    for b in range(_QR - 1):
        issue(b, b)

    def body(j, carry):
        s = lax.rem(j, _QR)
        pltpu.make_async_copy(p_hbm.at[ridx_v.at[j]],
                              buf.at[s, pl.ds(0, CHUNK)], ga.at[s]).wait()
        pltpu.make_async_copy(pneg_hbm.at[cidx_v.at[j]],
                              buf.at[s, pl.ds(CHUNK, CHUNK)], gb.at[s]).wait()

        def row_body(r, c2):
            for c in range(D // 16):
                sl = pl.ds(c * 16, 16)
                plsc.addupdate(buf.at[s, r, sl], buf[s, CHUNK + r, sl])
            return c2

        lax.fori_loop(0, CHUNK, row_body, 0)
        pltpu.async_copy(buf.at[s, pl.ds(0, CHUNK)],
                         q_hbm.at[pl.ds(base + j * CHUNK, CHUNK)], wsem.at[s])
        s2 = lax.rem(j + _QR - 1, _QR)

        @pl.when(jnp.logical_and(j >= 1, j + _QR - 1 < E_CH))
        def _():
            pltpu.make_async_copy(
                buf.at[s2, pl.ds(0, CHUNK)],
                q_hbm.at[pl.ds(base + (j - 1) * CHUNK, CHUNK)],
                wsem.at[s2]).wait()

        @pl.when(j + _QR - 1 < E_CH)
        def _():
            issue(j + _QR - 1, s2)
        return carry

    lax.fori_loop(0, E_CH, body, 0)

    def drain(j, carry):
        s = lax.rem(j, _QR)
        pltpu.make_async_copy(buf.at[s, pl.ds(0, CHUNK)],
                              q_hbm.at[pl.ds(base + j * CHUNK, CHUNK)],
                              wsem.at[s]).wait()
        return carry

    lax.fori_loop(E_CH - _QR, E_CH, drain, 0)


BLK = 1024
EBLK = 2048


def _k1_body(x_ref, w1_ref, degp_ref, z1_ref, dinv_ref):
    deg = degp_ref[:, 0:1] + degp_ref[:, 1:2] + 1.0
    dinv = lax.rsqrt(deg)
    y = jnp.dot(x_ref[...], w1_ref[...], preferred_element_type=jnp.float32)
    z1_ref[...] = y * dinv
    dinv_ref[...] = dinv


def _k2_body(agg_ref, z1_ref, dinv_ref, b1_ref, w2_ref, z2_ref):
    agg = agg_ref[0] + agg_ref[1]
    dinv = dinv_ref[...]
    f1 = jnp.maximum((agg + z1_ref[...]) * dinv + b1_ref[...], 0.0)
    z2_ref[...] = jnp.dot(f1, w2_ref[...],
                          preferred_element_type=jnp.float32) * dinv


def _k3_body(agg_ref, z2_ref, dinv_ref, b2_ref, m1_ref, p_ref, pneg_ref):
    agg = agg_ref[0] + agg_ref[1]
    dinv = dinv_ref[...]
    f2 = jnp.maximum((agg + z2_ref[...]) * dinv + b2_ref[...], 0.0)
    p = jnp.dot(f2, m1_ref[...], preferred_element_type=jnp.float32)
    p_ref[...] = p
    pneg_ref[...] = -p


def _k4_body(q_ref, c1_ref, m2_ref, c2_ref, m3_ref, c3_ref, o_ref):
    h1 = jnp.maximum(q_ref[...] + c1_ref[...], 0.0)
    h2 = jnp.maximum(
        jnp.dot(h1, m2_ref[...], preferred_element_type=jnp.float32)
        + c2_ref[...], 0.0)
    o = jnp.dot(h2, m3_ref[...], preferred_element_type=jnp.float32)
    o_ref[...] = jax.nn.sigmoid(o + c3_ref[...])


def _row_spec(i):
    return (i, 0)


def _rep_spec(i):
    return (0, 0)


def _node_spec(i):
    return (0, i, 0)


def _aggregate(z, src_r, dstA_r, dstB_r, zerosA):
    aggA = _agg_kernel(z, src_r, dstA_r, zerosA)
    aggB = _agg_kernel(z, src_r, dstB_r, zerosA)
    return jnp.concatenate([aggA[:, :SPLIT], aggB[:, :SPLIT]], axis=1)


def kernel(x, edge_index, W1, b1, W2, b2, M1, c1, M2, c2, M3, c3):
    f32 = jnp.float32
    ei = edge_index.astype(jnp.int32)
    row, col = ei[:, 0], ei[:, 1]
    fill = jnp.full((A_PAD - 2 * E,), NP_ - 1, jnp.int32)
    src_u = jnp.concatenate([row, col, fill])
    dst_u = jnp.concatenate([col, row, fill])
    inA = dst_u < SPLIT
    src_r = src_u.reshape(NW, A_G * AGG_C)
    dstA_r = jnp.where(inA, dst_u, SPLIT).reshape(NW, A_G * AGG_C)
    dstB_r = jnp.where(inA, SPLIT, dst_u - SPLIT).reshape(NW, A_G * AGG_C)
    efill = jnp.zeros((E_PAD - E,), jnp.int32)
    row_r = jnp.concatenate([row, efill]).reshape(NW, E_CH, CHUNK)
    col_r = jnp.concatenate([col, efill]).reshape(NW, E_CH, CHUNK)

    zerosA = jnp.zeros((ACC_R, D), f32)
    zeros1d = jnp.zeros((NP_,), f32)
    ones1d = jnp.ones((CHUNK,), f32)
    x_pad = jnp.concatenate([x, jnp.zeros((NP_ - N, D), f32)], axis=0)

    degp = _deg_kernel(dst_r := dst_u.reshape(NW, A_CH, CHUNK), ones1d,
                       zeros1d)
    degp_t = degp.reshape(NC, NP_).T

    grid_n = NP_ // BLK
    z1, dinv = pl.pallas_call(
        _k1_body,
        grid=(grid_n,),
        in_specs=[
            pl.BlockSpec((BLK, D), _row_spec),
            pl.BlockSpec((D, D), _rep_spec),
            pl.BlockSpec((BLK, NC), _row_spec),
        ],
        out_specs=[
            pl.BlockSpec((BLK, D), _row_spec),
            pl.BlockSpec((BLK, 1), _row_spec),
        ],
        out_shape=[
            jax.ShapeDtypeStruct((NP_, D), f32),
            jax.ShapeDtypeStruct((NP_, 1), f32),
        ],
    )(x_pad, W1, degp_t)

    agg1 = _aggregate(z1, src_r, dstA_r, dstB_r, zerosA)

    z2 = pl.pallas_call(
        _k2_body,
        grid=(grid_n,),
        in_specs=[
            pl.BlockSpec((NC, BLK, D), _node_spec),
            pl.BlockSpec((BLK, D), _row_spec),
            pl.BlockSpec((BLK, 1), _row_spec),
            pl.BlockSpec((1, D), _rep_spec),
            pl.BlockSpec((D, D), _rep_spec),
        ],
        out_specs=pl.BlockSpec((BLK, D), _row_spec),
        out_shape=jax.ShapeDtypeStruct((NP_, D), f32),
    )(agg1, z1, dinv, b1.reshape(1, D), W2)

    agg2 = _aggregate(z2, src_r, dstA_r, dstB_r, zerosA)

    p, pneg = pl.pallas_call(
        _k3_body,
        grid=(grid_n,),
        in_specs=[
            pl.BlockSpec((NC, BLK, D), _node_spec),
            pl.BlockSpec((BLK, D), _row_spec),
            pl.BlockSpec((BLK, 1), _row_spec),
            pl.BlockSpec((1, D), _rep_spec),
            pl.BlockSpec((D, D), _rep_spec),
        ],
        out_specs=[
            pl.BlockSpec((BLK, D), _row_spec),
            pl.BlockSpec((BLK, D), _row_spec),
        ],
        out_shape=[
            jax.ShapeDtypeStruct((NP_, D), f32),
            jax.ShapeDtypeStruct((NP_, D), f32),
        ],
    )(agg2, z2, dinv, b2.reshape(1, D), M1)

    q = _q_kernel(p, pneg, row_r, col_r)

    m3p = jnp.concatenate([M3, jnp.zeros((D, 7), f32)], axis=1)
    c3t = jnp.broadcast_to(c3.reshape(1, 1), (1, 8))
    out = pl.pallas_call(
        _k4_body,
        grid=(E_PAD // EBLK,),
        in_specs=[
            pl.BlockSpec((EBLK, D), _row_spec),
            pl.BlockSpec((1, D), _rep_spec),
            pl.BlockSpec((D, D), _rep_spec),
            pl.BlockSpec((1, D), _rep_spec),
            pl.BlockSpec((D, 8), _rep_spec),
            pl.BlockSpec((1, 8), _rep_spec),
        ],
        out_specs=pl.BlockSpec((EBLK, 8), _row_spec),
        out_shape=jax.ShapeDtypeStruct((E_PAD, 8), f32),
    )(q, c1.reshape(1, D), M2, c2.reshape(1, D), m3p, c3t)

    return out[:E, :1]


# speedup vs baseline: 10.3670x; 1.0044x over previous
import functools

import jax
import jax.numpy as jnp
from jax import lax
from jax.experimental import pallas as pl
from jax.experimental.pallas import tpu as pltpu
from jax.experimental.pallas import tpu_sc as plsc

N = 10000
E = 320000
D = 128
NC, NS = 2, 16
NW = NC * NS
CHUNK = 128

NP_ = 10240
RPT = NP_ // NW
SPLIT = 5120
ACC_R = 5376
RPT_A = ACC_R // NW
A_CH = 160
A_PAD = NW * A_CH * CHUNK
E_CH = 80
E_PAD = NW * E_CH * CHUNK

_mesh = plsc.VectorSubcoreMesh(
    core_axis_name="c", subcore_axis_name="s", num_cores=NC, num_subcores=NS)


_DEG_K = 8


@functools.partial(
    pl.kernel,
    out_type=jax.ShapeDtypeStruct((NC * NP_,), jnp.float32),
    mesh=_mesh,
    scratch_types=[
        pltpu.VMEM((A_CH, CHUNK), jnp.int32),
        pltpu.VMEM((CHUNK,), jnp.float32),
        pltpu.VMEM((RPT,), jnp.float32),
        pltpu.VMEM_SHARED((NP_,), jnp.float32),
        pltpu.SemaphoreType.DMA,
    ],
)
def _deg_kernel(dst_hbm, ones_hbm, zeros_hbm, degp_hbm, idx_v, ones_v, db_v,
                acc_sh, dsem):
    cid = lax.axis_index("c")
    sid = lax.axis_index("s")
    wid = sid * NC + cid
    r0 = sid * RPT
    pltpu.sync_copy(ones_hbm, ones_v)
    pltpu.sync_copy(zeros_hbm.at[pl.ds(r0, RPT)], db_v)
    pltpu.sync_copy(db_v, acc_sh.at[pl.ds(r0, RPT)])
    pltpu.sync_copy(dst_hbm.at[wid], idx_v)
    plsc.subcore_barrier()

    def body(j, carry):
        pltpu.async_copy(ones_v, acc_sh.at[idx_v.at[j]], dsem, add=True)

        @pl.when(j >= _DEG_K)
        def _():
            pltpu.make_async_copy(ones_v, acc_sh.at[idx_v.at[j - _DEG_K]],
                                  dsem).wait()
        return carry

    lax.fori_loop(0, A_CH, body, 0)

    def drain(j, carry):
        pltpu.make_async_copy(ones_v, acc_sh.at[idx_v.at[j]], dsem).wait()
        return carry

    lax.fori_loop(A_CH - _DEG_K, A_CH, drain, 0)
    plsc.subcore_barrier()
    pltpu.sync_copy(acc_sh.at[pl.ds(r0, RPT)], db_v)
    pltpu.sync_copy(db_v, degp_hbm.at[pl.ds(cid * NP_ + r0, RPT)])


AGG_C = 320
A_G = A_PAD // (NW * AGG_C)


_C0 = 64
_C1 = 64
_CMAX = max(_C0, _C1)


@functools.partial(
    pl.kernel,
    out_type=jax.ShapeDtypeStruct((NC, ACC_R, D), jnp.float32),
    mesh=_mesh,
    scratch_types=[
        pltpu.VMEM((_CMAX * AGG_C,), jnp.int32),
        pltpu.VMEM((_CMAX * AGG_C,), jnp.int32),
        pltpu.VMEM((AGG_C, D), jnp.float32),
        pltpu.VMEM_SHARED((ACC_R, D), jnp.float32),
    ],
)
def _agg_kernel(z_hbm, src_hbm, dst_hbm, zeros_hbm, aggp_hbm,
                sidx_v, didx_v, rows_v, acc_sh):
    cid = lax.axis_index("c")
    sid = lax.axis_index("s")
    wid = sid * NC + cid
    r0 = sid * RPT_A
    cnt = jnp.where(cid == 0, _C0, _C1)
    pltpu.sync_copy(src_hbm.at[wid], sidx_v)
    pltpu.sync_copy(dst_hbm.at[wid], didx_v)
    pltpu.sync_copy(zeros_hbm.at[pl.ds(r0, RPT_A)],
                    rows_v.at[pl.ds(0, RPT_A)])
    pltpu.sync_copy(rows_v.at[pl.ds(0, RPT_A)], acc_sh.at[pl.ds(r0, RPT_A)])
    plsc.subcore_barrier()

    def body(g, carry):
        sl = pl.ds(g * AGG_C, AGG_C)
        pltpu.sync_copy(z_hbm.at[sidx_v.at[sl]], rows_v)
        pltpu.sync_copy(rows_v, acc_sh.at[didx_v.at[sl]], add=True)
        return carry

    lax.fori_loop(0, cnt, body, 0)
    plsc.subcore_barrier()
    pltpu.sync_copy(acc_sh.at[pl.ds(r0, RPT_A)], rows_v.at[pl.ds(0, RPT_A)])
    pltpu.sync_copy(rows_v.at[pl.ds(0, RPT_A)],
                    aggp_hbm.at[cid, pl.ds(r0, RPT_A)])


_QR = 3


@functools.partial(
    pl.kernel,
    out_type=jax.ShapeDtypeStruct((E_PAD, D), jnp.float32),
    mesh=_mesh,
    scratch_types=[
        pltpu.VMEM((E_CH, CHUNK), jnp.int32),
        pltpu.VMEM((E_CH, CHUNK), jnp.int32),
        pltpu.VMEM((_QR, 2 * CHUNK, D), jnp.float32),
        pltpu.SemaphoreType.DMA((_QR,)),
        pltpu.SemaphoreType.DMA((_QR,)),
        pltpu.SemaphoreType.DMA((_QR,)),
    ],
)
def _q_kernel(p_hbm, pneg_hbm, row_hbm, col_hbm, q_hbm,
              ridx_v, cidx_v, buf, ga, gb, wsem):
    cid = lax.axis_index("c")
    sid = lax.axis_index("s")
    wid = sid * NC + cid
    base = wid * E_CH * CHUNK
    pltpu.sync_copy(row_hbm.at[wid], ridx_v)
    pltpu.sync_copy(col_hbm.at[wid], cidx_v)

    def issue(j, s):
        pltpu.async_copy(p_hbm.at[ridx_v.at[j]],
                         buf.at[s, pl.ds(0, CHUNK)], ga.at[s])
        pltpu.async_copy(pneg_hbm.at[cidx_v.at[j]],
                         buf.at[s, pl.ds(CHUNK, CHUNK)], gb.at[s])

    for b in range(_QR - 1):
        issue(b, b)

    def body(j, carry):
        s = lax.rem(j, _QR)
        pltpu.make_async_copy(p_hbm.at[ridx_v.at[j]],
                              buf.at[s, pl.ds(0, CHUNK)], ga.at[s]).wait()
        pltpu.make_async_copy(pneg_hbm.at[cidx_v.at[j]],
                              buf.at[s, pl.ds(CHUNK, CHUNK)], gb.at[s]).wait()

        def row_body(r, c2):
            for c in range(D // 16):
                sl = pl.ds(c * 16, 16)
                plsc.addupdate(buf.at[s, r, sl], buf[s, CHUNK + r, sl])
            return c2

        lax.fori_loop(0, CHUNK, row_body, 0)
        pltpu.async_copy(buf.at[s, pl.ds(0, CHUNK)],
                         q_hbm.at[pl.ds(base + j * CHUNK, CHUNK)], wsem.at[s])
        s2 = lax.rem(j + _QR - 1, _QR)

        @pl.when(jnp.logical_and(j >= 1, j + _QR - 1 < E_CH))
        def _():
            pltpu.make_async_copy(
                buf.at[s2, pl.ds(0, CHUNK)],
                q_hbm.at[pl.ds(base + (j - 1) * CHUNK, CHUNK)],
                wsem.at[s2]).wait()

        @pl.when(j + _QR - 1 < E_CH)
        def _():
            issue(j + _QR - 1, s2)
        return carry

    lax.fori_loop(0, E_CH, body, 0)

    def drain(j, carry):
        s = lax.rem(j, _QR)
        pltpu.make_async_copy(buf.at[s, pl.ds(0, CHUNK)],
                              q_hbm.at[pl.ds(base + j * CHUNK, CHUNK)],
                              wsem.at[s]).wait()
        return carry

    lax.fori_loop(E_CH - _QR, E_CH, drain, 0)


BLK = 1024
EBLK = 2048


def _k1_body(x_ref, w1_ref, degp_ref, z1_ref, dinv_ref):
    deg = degp_ref[:, 0:1] + degp_ref[:, 1:2] + 1.0
    dinv = lax.rsqrt(deg)
    y = jnp.dot(x_ref[...], w1_ref[...], preferred_element_type=jnp.float32)
    z1_ref[...] = y * dinv
    dinv_ref[...] = dinv


def _k2_body(agg_ref, z1_ref, dinv_ref, b1_ref, w2_ref, z2_ref):
    agg = agg_ref[0] + agg_ref[1]
    dinv = dinv_ref[...]
    f1 = jnp.maximum((agg + z1_ref[...]) * dinv + b1_ref[...], 0.0)
    z2_ref[...] = jnp.dot(f1, w2_ref[...],
                          preferred_element_type=jnp.float32) * dinv


def _k3_body(agg_ref, z2_ref, dinv_ref, b2_ref, m1_ref, p_ref, pneg_ref):
    agg = agg_ref[0] + agg_ref[1]
    dinv = dinv_ref[...]
    f2 = jnp.maximum((agg + z2_ref[...]) * dinv + b2_ref[...], 0.0)
    p = jnp.dot(f2, m1_ref[...], preferred_element_type=jnp.float32)
    p_ref[...] = p
    pneg_ref[...] = -p


def _k4_body(q_ref, c1_ref, m2_ref, c2_ref, m3_ref, c3_ref, o_ref):
    h1 = jnp.maximum(q_ref[...] + c1_ref[...], 0.0)
    h2 = jnp.maximum(
        jnp.dot(h1, m2_ref[...], preferred_element_type=jnp.float32)
        + c2_ref[...], 0.0)
    o = jnp.dot(h2, m3_ref[...], preferred_element_type=jnp.float32)
    o_ref[...] = jax.nn.sigmoid(o + c3_ref[...])


def _row_spec(i):
    return (i, 0)


def _rep_spec(i):
    return (0, 0)


def _node_spec(i):
    return (0, i, 0)


def _aggregate(z, src_r, dstA_r, dstB_r, zerosA):
    aggA = _agg_kernel(z, src_r, dstA_r, zerosA)
    aggB = _agg_kernel(z, src_r, dstB_r, zerosA)
    return jnp.concatenate([aggA[:, :SPLIT], aggB[:, :SPLIT]], axis=1)


def kernel(x, edge_index, W1, b1, W2, b2, M1, c1, M2, c2, M3, c3):
    f32 = jnp.float32
    ei = edge_index.astype(jnp.int32)
    row, col = ei[:, 0], ei[:, 1]
    fill = jnp.full((A_PAD - 2 * E,), NP_ - 1, jnp.int32)
    src_u = jnp.concatenate([row, col, fill])
    dst_u = jnp.concatenate([col, row, fill])
    inA = dst_u < SPLIT
    def per_core_rows(flat, pad_val):
        l0 = 16 * _C0 * AGG_C
        rows0 = flat[:l0].reshape(16, _C0 * AGG_C)
        rows1 = flat[l0:].reshape(16, _C1 * AGG_C)
        cm = _CMAX * AGG_C
        rows0 = jnp.pad(rows0, ((0, 0), (0, cm - _C0 * AGG_C)),
                        constant_values=pad_val)
        rows1 = jnp.pad(rows1, ((0, 0), (0, cm - _C1 * AGG_C)),
                        constant_values=pad_val)
        return jnp.stack([rows0, rows1], axis=1).reshape(NW, cm)

    src_r = per_core_rows(src_u, NP_ - 1)
    dstA_r = per_core_rows(jnp.where(inA, dst_u, SPLIT), SPLIT)
    dstB_r = per_core_rows(jnp.where(inA, SPLIT, dst_u - SPLIT), SPLIT)
    efill = jnp.zeros((E_PAD - E,), jnp.int32)
    row_r = jnp.concatenate([row, efill]).reshape(NW, E_CH, CHUNK)
    col_r = jnp.concatenate([col, efill]).reshape(NW, E_CH, CHUNK)

    zerosA = jnp.zeros((ACC_R, D), f32)
    zeros1d = jnp.zeros((NP_,), f32)
    ones1d = jnp.ones((CHUNK,), f32)
    x_pad = jnp.concatenate([x, jnp.zeros((NP_ - N, D), f32)], axis=0)

    degp = _deg_kernel(dst_r := dst_u.reshape(NW, A_CH, CHUNK), ones1d,
                       zeros1d)
    degp_t = degp.reshape(NC, NP_).T

    grid_n = NP_ // BLK
    z1, dinv = pl.pallas_call(
        _k1_body,
        grid=(grid_n,),
        in_specs=[
            pl.BlockSpec((BLK, D), _row_spec),
            pl.BlockSpec((D, D), _rep_spec),
            pl.BlockSpec((BLK, NC), _row_spec),
        ],
        out_specs=[
            pl.BlockSpec((BLK, D), _row_spec),
            pl.BlockSpec((BLK, 1), _row_spec),
        ],
        out_shape=[
            jax.ShapeDtypeStruct((NP_, D), f32),
            jax.ShapeDtypeStruct((NP_, 1), f32),
        ],
    )(x_pad, W1, degp_t)

    agg1 = _aggregate(z1, src_r, dstA_r, dstB_r, zerosA)

    z2 = pl.pallas_call(
        _k2_body,
        grid=(grid_n,),
        in_specs=[
            pl.BlockSpec((NC, BLK, D), _node_spec),
            pl.BlockSpec((BLK, D), _row_spec),
            pl.BlockSpec((BLK, 1), _row_spec),
            pl.BlockSpec((1, D), _rep_spec),
            pl.BlockSpec((D, D), _rep_spec),
        ],
        out_specs=pl.BlockSpec((BLK, D), _row_spec),
        out_shape=jax.ShapeDtypeStruct((NP_, D), f32),
    )(agg1, z1, dinv, b1.reshape(1, D), W2)

    agg2 = _aggregate(z2, src_r, dstA_r, dstB_r, zerosA)

    p, pneg = pl.pallas_call(
        _k3_body,
        grid=(grid_n,),
        in_specs=[
            pl.BlockSpec((NC, BLK, D), _node_spec),
            pl.BlockSpec((BLK, D), _row_spec),
            pl.BlockSpec((BLK, 1), _row_spec),
            pl.BlockSpec((1, D), _rep_spec),
            pl.BlockSpec((D, D), _rep_spec),
        ],
        out_specs=[
            pl.BlockSpec((BLK, D), _row_spec),
            pl.BlockSpec((BLK, D), _row_spec),
        ],
        out_shape=[
            jax.ShapeDtypeStruct((NP_, D), f32),
            jax.ShapeDtypeStruct((NP_, D), f32),
        ],
    )(agg2, z2, dinv, b2.reshape(1, D), M1)

    q = _q_kernel(p, pneg, row_r, col_r)

    m3p = jnp.concatenate([M3, jnp.zeros((D, 7), f32)], axis=1)
    c3t = jnp.broadcast_to(c3.reshape(1, 1), (1, 8))
    out = pl.pallas_call(
        _k4_body,
        grid=(E_PAD // EBLK,),
        in_specs=[
            pl.BlockSpec((EBLK, D), _row_spec),
            pl.BlockSpec((1, D), _rep_spec),
            pl.BlockSpec((D, D), _rep_spec),
            pl.BlockSpec((1, D), _rep_spec),
            pl.BlockSpec((D, 8), _rep_spec),
            pl.BlockSpec((1, 8), _rep_spec),
        ],
        out_specs=pl.BlockSpec((EBLK, 8), _row_spec),
        out_shape=jax.ShapeDtypeStruct((E_PAD, 8), f32),
    )(q, c1.reshape(1, D), M2, c2.reshape(1, D), m3p, c3t)

    return out[:E, :1]


# speedup vs baseline: 10.3694x; 1.0002x over previous
import functools

import jax
import jax.numpy as jnp
from jax import lax
from jax.experimental import pallas as pl
from jax.experimental.pallas import tpu as pltpu
from jax.experimental.pallas import tpu_sc as plsc

N = 10000
E = 320000
D = 128
NC, NS = 2, 16
NW = NC * NS
CHUNK = 128

NP_ = 10240
RPT = NP_ // NW
SPLIT = 5120
ACC_R = 5376
RPT_A = ACC_R // NW
A_CH = 160
A_PAD = NW * A_CH * CHUNK
E_CH = 80
E_PAD = NW * E_CH * CHUNK

_mesh = plsc.VectorSubcoreMesh(
    core_axis_name="c", subcore_axis_name="s", num_cores=NC, num_subcores=NS)


_DEG_K = 8


@functools.partial(
    pl.kernel,
    out_type=jax.ShapeDtypeStruct((NC * NP_,), jnp.float32),
    mesh=_mesh,
    scratch_types=[
        pltpu.VMEM((A_CH, CHUNK), jnp.int32),
        pltpu.VMEM((CHUNK,), jnp.float32),
        pltpu.VMEM((RPT,), jnp.float32),
        pltpu.VMEM_SHARED((NP_,), jnp.float32),
        pltpu.SemaphoreType.DMA,
    ],
)
def _deg_kernel(dst_hbm, ones_hbm, zeros_hbm, degp_hbm, idx_v, ones_v, db_v,
                acc_sh, dsem):
    cid = lax.axis_index("c")
    sid = lax.axis_index("s")
    wid = sid * NC + cid
    r0 = sid * RPT
    pltpu.sync_copy(ones_hbm, ones_v)
    pltpu.sync_copy(zeros_hbm.at[pl.ds(r0, RPT)], db_v)
    pltpu.sync_copy(db_v, acc_sh.at[pl.ds(r0, RPT)])
    pltpu.sync_copy(dst_hbm.at[wid], idx_v)
    plsc.subcore_barrier()

    def body(j, carry):
        pltpu.async_copy(ones_v, acc_sh.at[idx_v.at[j]], dsem, add=True)

        @pl.when(j >= _DEG_K)
        def _():
            pltpu.make_async_copy(ones_v, acc_sh.at[idx_v.at[j - _DEG_K]],
                                  dsem).wait()
        return carry

    lax.fori_loop(0, A_CH, body, 0)

    def drain(j, carry):
        pltpu.make_async_copy(ones_v, acc_sh.at[idx_v.at[j]], dsem).wait()
        return carry

    lax.fori_loop(A_CH - _DEG_K, A_CH, drain, 0)
    plsc.subcore_barrier()
    pltpu.sync_copy(acc_sh.at[pl.ds(r0, RPT)], db_v)
    pltpu.sync_copy(db_v, degp_hbm.at[pl.ds(cid * NP_ + r0, RPT)])


AGG_C = 320
A_G = A_PAD // (NW * AGG_C)


_C0 = 64
_C1 = 64
_CMAX = max(_C0, _C1)


@functools.partial(
    pl.kernel,
    out_type=jax.ShapeDtypeStruct((NC, ACC_R, D), jnp.float32),
    mesh=_mesh,
    scratch_types=[
        pltpu.VMEM((_CMAX * AGG_C,), jnp.int32),
        pltpu.VMEM((_CMAX * AGG_C,), jnp.int32),
        pltpu.VMEM((AGG_C, D), jnp.float32),
        pltpu.VMEM_SHARED((ACC_R, D), jnp.float32),
    ],
)
def _agg_kernel(z_hbm, src_hbm, dst_hbm, zeros_hbm, aggp_hbm,
                sidx_v, didx_v, rows_v, acc_sh):
    cid = lax.axis_index("c")
    sid = lax.axis_index("s")
    wid = sid * NC + cid
    r0 = sid * RPT_A
    cnt = jnp.where(cid == 0, _C0, _C1)
    pltpu.sync_copy(src_hbm.at[wid], sidx_v)
    pltpu.sync_copy(dst_hbm.at[wid], didx_v)
    pltpu.sync_copy(zeros_hbm.at[pl.ds(r0, RPT_A)],
                    rows_v.at[pl.ds(0, RPT_A)])
    pltpu.sync_copy(rows_v.at[pl.ds(0, RPT_A)], acc_sh.at[pl.ds(r0, RPT_A)])
    plsc.subcore_barrier()

    def body(g, carry):
        sl = pl.ds(g * AGG_C, AGG_C)
        pltpu.sync_copy(z_hbm.at[sidx_v.at[sl]], rows_v)
        pltpu.sync_copy(rows_v, acc_sh.at[didx_v.at[sl]], add=True)
        return carry

    lax.fori_loop(0, cnt, body, 0)
    plsc.subcore_barrier()
    pltpu.sync_copy(acc_sh.at[pl.ds(r0, RPT_A)], rows_v.at[pl.ds(0, RPT_A)])
    pltpu.sync_copy(rows_v.at[pl.ds(0, RPT_A)],
                    aggp_hbm.at[cid, pl.ds(r0, RPT_A)])


_QR = 3


@functools.partial(
    pl.kernel,
    out_type=jax.ShapeDtypeStruct((E_PAD, D), jnp.float32),
    mesh=_mesh,
    scratch_types=[
        pltpu.VMEM((E_CH, CHUNK), jnp.int32),
        pltpu.VMEM((E_CH, CHUNK), jnp.int32),
        pltpu.VMEM((_QR, 2 * CHUNK, D), jnp.float32),
        pltpu.SemaphoreType.DMA((_QR,)),
        pltpu.SemaphoreType.DMA((_QR,)),
        pltpu.SemaphoreType.DMA((_QR,)),
    ],
)
def _q_kernel(p_hbm, pneg_hbm, row_hbm, col_hbm, q_hbm,
              ridx_v, cidx_v, buf, ga, gb, wsem):
    cid = lax.axis_index("c")
    sid = lax.axis_index("s")
    wid = sid * NC + cid
    base = wid * E_CH * CHUNK
    pltpu.sync_copy(row_hbm.at[wid], ridx_v)
    pltpu.sync_copy(col_hbm.at[wid], cidx_v)

    def issue(j, s):
        pltpu.async_copy(p_hbm.at[ridx_v.at[j]],
                         buf.at[s, pl.ds(0, CHUNK)], ga.at[s])
        pltpu.async_copy(pneg_hbm.at[cidx_v.at[j]],
                         buf.at[s, pl.ds(CHUNK, CHUNK)], gb.at[s])

    for b in range(_QR - 1):
        issue(b, b)

    def body(j, carry):
        s = lax.rem(j, _QR)
        pltpu.make_async_copy(p_hbm.at[ridx_v.at[j]],
                              buf.at[s, pl.ds(0, CHUNK)], ga.at[s]).wait()
        pltpu.make_async_copy(pneg_hbm.at[cidx_v.at[j]],
                              buf.at[s, pl.ds(CHUNK, CHUNK)], gb.at[s]).wait()

        def row_body(r, c2):
            for c in range(D // 16):
                sl = pl.ds(c * 16, 16)
                plsc.addupdate(buf.at[s, r, sl], buf[s, CHUNK + r, sl])
            return c2

        lax.fori_loop(0, CHUNK, row_body, 0)
        pltpu.async_copy(buf.at[s, pl.ds(0, CHUNK)],
                         q_hbm.at[pl.ds(base + j * CHUNK, CHUNK)], wsem.at[s])
        s2 = lax.rem(j + _QR - 1, _QR)

        @pl.when(jnp.logical_and(j >= 1, j + _QR - 1 < E_CH))
        def _():
            pltpu.make_async_copy(
                buf.at[s2, pl.ds(0, CHUNK)],
                q_hbm.at[pl.ds(base + (j - 1) * CHUNK, CHUNK)],
                wsem.at[s2]).wait()

        @pl.when(j + _QR - 1 < E_CH)
        def _():
            issue(j + _QR - 1, s2)
        return carry

    lax.fori_loop(0, E_CH, body, 0)

    def drain(j, carry):
        s = lax.rem(j, _QR)
        pltpu.make_async_copy(buf.at[s, pl.ds(0, CHUNK)],
                              q_hbm.at[pl.ds(base + j * CHUNK, CHUNK)],
                              wsem.at[s]).wait()
        return carry

    lax.fori_loop(E_CH - _QR, E_CH, drain, 0)


BLK = 1024
EBLK = 2048


def _k1_body(x_ref, w1_ref, degp_ref, z1_ref, dinv_ref):
    deg = degp_ref[:, 0:1] + degp_ref[:, 1:2] + 1.0
    dinv = lax.rsqrt(deg)
    y = jnp.dot(x_ref[...], w1_ref[...], preferred_element_type=jnp.float32)
    z1_ref[...] = y * dinv
    dinv_ref[...] = dinv


def _k2_body(agg_ref, z1_ref, dinv_ref, b1_ref, w2_ref, z2_ref):
    agg = agg_ref[0] + agg_ref[1]
    dinv = dinv_ref[...]
    f1 = jnp.maximum((agg + z1_ref[...]) * dinv + b1_ref[...], 0.0)
    z2_ref[...] = jnp.dot(f1, w2_ref[...],
                          preferred_element_type=jnp.float32) * dinv


def _k3_body(agg_ref, z2_ref, dinv_ref, b2_ref, m1_ref, p_ref, pneg_ref):
    agg = agg_ref[0] + agg_ref[1]
    dinv = dinv_ref[...]
    f2 = jnp.maximum((agg + z2_ref[...]) * dinv + b2_ref[...], 0.0)
    p = jnp.dot(f2, m1_ref[...], preferred_element_type=jnp.float32)
    p_ref[...] = p
    pneg_ref[...] = -p


def _k4_body(q_ref, c1_ref, m2_ref, c2_ref, m3_ref, c3_ref, o_ref):
    h1 = jnp.maximum(q_ref[...] + c1_ref[...], 0.0)
    h2 = jnp.maximum(
        jnp.dot(h1.astype(jnp.bfloat16), m2_ref[...].astype(jnp.bfloat16),
                preferred_element_type=jnp.float32)
        + c2_ref[...], 0.0)
    o = jnp.dot(h2.astype(jnp.bfloat16), m3_ref[...].astype(jnp.bfloat16),
                preferred_element_type=jnp.float32)
    o_ref[...] = jax.nn.sigmoid(o + c3_ref[...])


def _row_spec(i):
    return (i, 0)


def _rep_spec(i):
    return (0, 0)


def _node_spec(i):
    return (0, i, 0)


def _aggregate(z, src_r, dstA_r, dstB_r, zerosA):
    aggA = _agg_kernel(z, src_r, dstA_r, zerosA)
    aggB = _agg_kernel(z, src_r, dstB_r, zerosA)
    return jnp.concatenate([aggA[:, :SPLIT], aggB[:, :SPLIT]], axis=1)


def kernel(x, edge_index, W1, b1, W2, b2, M1, c1, M2, c2, M3, c3):
    f32 = jnp.float32
    ei = edge_index.astype(jnp.int32)
    row, col = ei[:, 0], ei[:, 1]
    fill = jnp.full((A_PAD - 2 * E,), NP_ - 1, jnp.int32)
    src_u = jnp.concatenate([row, col, fill])
    dst_u = jnp.concatenate([col, row, fill])
    inA = dst_u < SPLIT
    def per_core_rows(flat, pad_val):
        l0 = 16 * _C0 * AGG_C
        rows0 = flat[:l0].reshape(16, _C0 * AGG_C)
        rows1 = flat[l0:].reshape(16, _C1 * AGG_C)
        cm = _CMAX * AGG_C
        rows0 = jnp.pad(rows0, ((0, 0), (0, cm - _C0 * AGG_C)),
                        constant_values=pad_val)
        rows1 = jnp.pad(rows1, ((0, 0), (0, cm - _C1 * AGG_C)),
                        constant_values=pad_val)
        return jnp.stack([rows0, rows1], axis=1).reshape(NW, cm)

    src_r = per_core_rows(src_u, NP_ - 1)
    dstA_r = per_core_rows(jnp.where(inA, dst_u, SPLIT), SPLIT)
    dstB_r = per_core_rows(jnp.where(inA, SPLIT, dst_u - SPLIT), SPLIT)
    efill = jnp.zeros((E_PAD - E,), jnp.int32)
    row_r = jnp.concatenate([row, efill]).reshape(NW, E_CH, CHUNK)
    col_r = jnp.concatenate([col, efill]).reshape(NW, E_CH, CHUNK)

    zerosA = jnp.zeros((ACC_R, D), f32)
    zeros1d = jnp.zeros((NP_,), f32)
    ones1d = jnp.ones((CHUNK,), f32)
    x_pad = jnp.concatenate([x, jnp.zeros((NP_ - N, D), f32)], axis=0)

    degp = _deg_kernel(dst_r := dst_u.reshape(NW, A_CH, CHUNK), ones1d,
                       zeros1d)
    degp_t = degp.reshape(NC, NP_).T

    grid_n = NP_ // BLK
    z1, dinv = pl.pallas_call(
        _k1_body,
        grid=(grid_n,),
        in_specs=[
            pl.BlockSpec((BLK, D), _row_spec),
            pl.BlockSpec((D, D), _rep_spec),
            pl.BlockSpec((BLK, NC), _row_spec),
        ],
        out_specs=[
            pl.BlockSpec((BLK, D), _row_spec),
            pl.BlockSpec((BLK, 1), _row_spec),
        ],
        out_shape=[
            jax.ShapeDtypeStruct((NP_, D), f32),
            jax.ShapeDtypeStruct((NP_, 1), f32),
        ],
    )(x_pad, W1, degp_t)

    agg1 = _aggregate(z1, src_r, dstA_r, dstB_r, zerosA)

    z2 = pl.pallas_call(
        _k2_body,
        grid=(grid_n,),
        in_specs=[
            pl.BlockSpec((NC, BLK, D), _node_spec),
            pl.BlockSpec((BLK, D), _row_spec),
            pl.BlockSpec((BLK, 1), _row_spec),
            pl.BlockSpec((1, D), _rep_spec),
            pl.BlockSpec((D, D), _rep_spec),
        ],
        out_specs=pl.BlockSpec((BLK, D), _row_spec),
        out_shape=jax.ShapeDtypeStruct((NP_, D), f32),
    )(agg1, z1, dinv, b1.reshape(1, D), W2)

    agg2 = _aggregate(z2, src_r, dstA_r, dstB_r, zerosA)

    p, pneg = pl.pallas_call(
        _k3_body,
        grid=(grid_n,),
        in_specs=[
            pl.BlockSpec((NC, BLK, D), _node_spec),
            pl.BlockSpec((BLK, D), _row_spec),
            pl.BlockSpec((BLK, 1), _row_spec),
            pl.BlockSpec((1, D), _rep_spec),
            pl.BlockSpec((D, D), _rep_spec),
        ],
        out_specs=[
            pl.BlockSpec((BLK, D), _row_spec),
            pl.BlockSpec((BLK, D), _row_spec),
        ],
        out_shape=[
            jax.ShapeDtypeStruct((NP_, D), f32),
            jax.ShapeDtypeStruct((NP_, D), f32),
        ],
    )(agg2, z2, dinv, b2.reshape(1, D), M1)

    q = _q_kernel(p, pneg, row_r, col_r)

    m3p = jnp.concatenate([M3, jnp.zeros((D, 7), f32)], axis=1)
    c3t = jnp.broadcast_to(c3.reshape(1, 1), (1, 8))
    out = pl.pallas_call(
        _k4_body,
        grid=(E_PAD // EBLK,),
        in_specs=[
            pl.BlockSpec((EBLK, D), _row_spec),
            pl.BlockSpec((1, D), _rep_spec),
            pl.BlockSpec((D, D), _rep_spec),
            pl.BlockSpec((1, D), _rep_spec),
            pl.BlockSpec((D, 8), _rep_spec),
            pl.BlockSpec((1, 8), _rep_spec),
        ],
        out_specs=pl.BlockSpec((EBLK, 8), _row_spec),
        out_shape=jax.ShapeDtypeStruct((E_PAD, 8), f32),
    )(q, c1.reshape(1, D), M2, c2.reshape(1, D), m3p, c3t)

    return out[:E, :1]


# speedup vs baseline: 10.6974x; 1.0316x over previous
import functools

import jax
import jax.numpy as jnp
from jax import lax
from jax.experimental import pallas as pl
from jax.experimental.pallas import tpu as pltpu
from jax.experimental.pallas import tpu_sc as plsc

N = 10000
E = 320000
D = 128
NC, NS = 2, 16
NW = NC * NS
CHUNK = 128

NP_ = 10240
RPT = NP_ // NW
SPLIT = 5120
ACC_R = 5376
RPT_A = ACC_R // NW
A_CH = 160
A_PAD = NW * A_CH * CHUNK
E_CH = 80
E_PAD = NW * E_CH * CHUNK

_mesh = plsc.VectorSubcoreMesh(
    core_axis_name="c", subcore_axis_name="s", num_cores=NC, num_subcores=NS)


_DEG_K = 8


@functools.partial(
    pl.kernel,
    out_type=jax.ShapeDtypeStruct((NC * NP_,), jnp.float32),
    mesh=_mesh,
    scratch_types=[
        pltpu.VMEM((A_CH, CHUNK), jnp.int32),
        pltpu.VMEM((CHUNK,), jnp.float32),
        pltpu.VMEM((RPT,), jnp.float32),
        pltpu.VMEM_SHARED((NP_,), jnp.float32),
        pltpu.SemaphoreType.DMA,
    ],
)
def _deg_kernel(dst_hbm, ones_hbm, zeros_hbm, degp_hbm, idx_v, ones_v, db_v,
                acc_sh, dsem):
    cid = lax.axis_index("c")
    sid = lax.axis_index("s")
    wid = sid * NC + cid
    r0 = sid * RPT
    pltpu.sync_copy(ones_hbm, ones_v)
    pltpu.sync_copy(zeros_hbm.at[pl.ds(r0, RPT)], db_v)
    pltpu.sync_copy(db_v, acc_sh.at[pl.ds(r0, RPT)])
    pltpu.sync_copy(dst_hbm.at[wid], idx_v)
    plsc.subcore_barrier()

    def body(j, carry):
        pltpu.async_copy(ones_v, acc_sh.at[idx_v.at[j]], dsem, add=True)

        @pl.when(j >= _DEG_K)
        def _():
            pltpu.make_async_copy(ones_v, acc_sh.at[idx_v.at[j - _DEG_K]],
                                  dsem).wait()
        return carry

    lax.fori_loop(0, A_CH, body, 0)

    def drain(j, carry):
        pltpu.make_async_copy(ones_v, acc_sh.at[idx_v.at[j]], dsem).wait()
        return carry

    lax.fori_loop(A_CH - _DEG_K, A_CH, drain, 0)
    plsc.subcore_barrier()
    pltpu.sync_copy(acc_sh.at[pl.ds(r0, RPT)], db_v)
    pltpu.sync_copy(db_v, degp_hbm.at[pl.ds(cid * NP_ + r0, RPT)])


AGG_C = 320
A_G = A_PAD // (NW * AGG_C)


_C0 = 64
_C1 = 64
_CMAX = max(_C0, _C1)


@functools.partial(
    pl.kernel,
    out_type=jax.ShapeDtypeStruct((NC, ACC_R, D), jnp.float32),
    mesh=_mesh,
    scratch_types=[
        pltpu.VMEM((_CMAX * AGG_C,), jnp.int32),
        pltpu.VMEM((_CMAX * AGG_C,), jnp.int32),
        pltpu.VMEM((AGG_C, D), jnp.float32),
        pltpu.VMEM_SHARED((ACC_R, D), jnp.float32),
    ],
)
def _agg_kernel(z_hbm, src_hbm, dst_hbm, zeros_hbm, aggp_hbm,
                sidx_v, didx_v, rows_v, acc_sh):
    cid = lax.axis_index("c")
    sid = lax.axis_index("s")
    wid = sid * NC + cid
    r0 = sid * RPT_A
    cnt = jnp.where(cid == 0, _C0, _C1)
    pltpu.sync_copy(src_hbm.at[wid], sidx_v)
    pltpu.sync_copy(dst_hbm.at[wid], didx_v)
    pltpu.sync_copy(zeros_hbm.at[pl.ds(r0, RPT_A)],
                    rows_v.at[pl.ds(0, RPT_A)])
    pltpu.sync_copy(rows_v.at[pl.ds(0, RPT_A)], acc_sh.at[pl.ds(r0, RPT_A)])
    plsc.subcore_barrier()

    def body(g, carry):
        sl = pl.ds(g * AGG_C, AGG_C)
        pltpu.sync_copy(z_hbm.at[sidx_v.at[sl]], rows_v)
        pltpu.sync_copy(rows_v, acc_sh.at[didx_v.at[sl]], add=True)
        return carry

    lax.fori_loop(0, cnt, body, 0)
    plsc.subcore_barrier()
    pltpu.sync_copy(acc_sh.at[pl.ds(r0, RPT_A)], rows_v.at[pl.ds(0, RPT_A)])
    pltpu.sync_copy(rows_v.at[pl.ds(0, RPT_A)],
                    aggp_hbm.at[cid, pl.ds(r0, RPT_A)])


_QR = 3
_Q0 = 48
_Q1 = 112
_QMAX = max(_Q0, _Q1)


@functools.partial(
    pl.kernel,
    out_type=jax.ShapeDtypeStruct((E_PAD, D), jnp.float32),
    mesh=_mesh,
    scratch_types=[
        pltpu.VMEM((_QMAX, CHUNK), jnp.int32),
        pltpu.VMEM((_QMAX, CHUNK), jnp.int32),
        pltpu.VMEM((_QR, 2 * CHUNK, D), jnp.float32),
        pltpu.SemaphoreType.DMA((_QR,)),
        pltpu.SemaphoreType.DMA((_QR,)),
        pltpu.SemaphoreType.DMA((_QR,)),
    ],
)
def _q_kernel(p_hbm, pneg_hbm, row_hbm, col_hbm, q_hbm,
              ridx_v, cidx_v, buf, ga, gb, wsem):
    cid = lax.axis_index("c")
    sid = lax.axis_index("s")
    wid = sid * NC + cid
    cnt = jnp.where(cid == 0, _Q0, _Q1)
    base = jnp.where(cid == 0, sid * _Q0, 16 * _Q0 + sid * _Q1) * CHUNK
    pltpu.sync_copy(row_hbm.at[wid], ridx_v)
    pltpu.sync_copy(col_hbm.at[wid], cidx_v)

    def issue(j, s):
        pltpu.async_copy(p_hbm.at[ridx_v.at[j]],
                         buf.at[s, pl.ds(0, CHUNK)], ga.at[s])
        pltpu.async_copy(pneg_hbm.at[cidx_v.at[j]],
                         buf.at[s, pl.ds(CHUNK, CHUNK)], gb.at[s])

    for b in range(_QR - 1):
        issue(b, b)

    def body(j, carry):
        s = lax.rem(j, _QR)
        pltpu.make_async_copy(p_hbm.at[ridx_v.at[j]],
                              buf.at[s, pl.ds(0, CHUNK)], ga.at[s]).wait()
        pltpu.make_async_copy(pneg_hbm.at[cidx_v.at[j]],
                              buf.at[s, pl.ds(CHUNK, CHUNK)], gb.at[s]).wait()

        def row_body(r, c2):
            for c in range(D // 16):
                sl = pl.ds(c * 16, 16)
                plsc.addupdate(buf.at[s, r, sl], buf[s, CHUNK + r, sl])
            return c2

        lax.fori_loop(0, CHUNK, row_body, 0)
        pltpu.async_copy(buf.at[s, pl.ds(0, CHUNK)],
                         q_hbm.at[pl.ds(base + j * CHUNK, CHUNK)], wsem.at[s])
        s2 = lax.rem(j + _QR - 1, _QR)

        @pl.when(jnp.logical_and(j >= 1, j + _QR - 1 < cnt))
        def _():
            pltpu.make_async_copy(
                buf.at[s2, pl.ds(0, CHUNK)],
                q_hbm.at[pl.ds(base + (j - 1) * CHUNK, CHUNK)],
                wsem.at[s2]).wait()

        @pl.when(j + _QR - 1 < cnt)
        def _():
            issue(j + _QR - 1, s2)
        return carry

    lax.fori_loop(0, cnt, body, 0)

    def drain(j, carry):
        s = lax.rem(j, _QR)
        pltpu.make_async_copy(buf.at[s, pl.ds(0, CHUNK)],
                              q_hbm.at[pl.ds(base + j * CHUNK, CHUNK)],
                              wsem.at[s]).wait()
        return carry

    lax.fori_loop(cnt - _QR, cnt, drain, 0)


BLK = 1024
EBLK = 2048


def _k1_body(x_ref, w1_ref, degp_ref, z1_ref, dinv_ref):
    deg = degp_ref[:, 0:1] + degp_ref[:, 1:2] + 1.0
    dinv = lax.rsqrt(deg)
    y = jnp.dot(x_ref[...], w1_ref[...], preferred_element_type=jnp.float32)
    z1_ref[...] = y * dinv
    dinv_ref[...] = dinv


def _k2_body(agg_ref, z1_ref, dinv_ref, b1_ref, w2_ref, z2_ref):
    agg = agg_ref[0] + agg_ref[1]
    dinv = dinv_ref[...]
    f1 = jnp.maximum((agg + z1_ref[...]) * dinv + b1_ref[...], 0.0)
    z2_ref[...] = jnp.dot(f1, w2_ref[...],
                          preferred_element_type=jnp.float32) * dinv


def _k3_body(agg_ref, z2_ref, dinv_ref, b2_ref, m1_ref, p_ref, pneg_ref):
    agg = agg_ref[0] + agg_ref[1]
    dinv = dinv_ref[...]
    f2 = jnp.maximum((agg + z2_ref[...]) * dinv + b2_ref[...], 0.0)
    p = jnp.dot(f2, m1_ref[...], preferred_element_type=jnp.float32)
    p_ref[...] = p
    pneg_ref[...] = -p


def _k4_body(q_ref, c1_ref, m2_ref, c2_ref, m3_ref, c3_ref, o_ref):
    h1 = jnp.maximum(q_ref[...] + c1_ref[...], 0.0)
    h2 = jnp.maximum(
        jnp.dot(h1.astype(jnp.bfloat16), m2_ref[...].astype(jnp.bfloat16),
                preferred_element_type=jnp.float32)
        + c2_ref[...], 0.0)
    o = jnp.dot(h2.astype(jnp.bfloat16), m3_ref[...].astype(jnp.bfloat16),
                preferred_element_type=jnp.float32)
    o_ref[...] = jax.nn.sigmoid(o + c3_ref[...])


def _row_spec(i):
    return (i, 0)


def _rep_spec(i):
    return (0, 0)


def _node_spec(i):
    return (0, i, 0)


def _aggregate(z, src_r, dstA_r, dstB_r, zerosA):
    aggA = _agg_kernel(z, src_r, dstA_r, zerosA)
    aggB = _agg_kernel(z, src_r, dstB_r, zerosA)
    return jnp.concatenate([aggA[:, :SPLIT], aggB[:, :SPLIT]], axis=1)


def kernel(x, edge_index, W1, b1, W2, b2, M1, c1, M2, c2, M3, c3):
    f32 = jnp.float32
    ei = edge_index.astype(jnp.int32)
    row, col = ei[:, 0], ei[:, 1]
    fill = jnp.full((A_PAD - 2 * E,), NP_ - 1, jnp.int32)
    src_u = jnp.concatenate([row, col, fill])
    dst_u = jnp.concatenate([col, row, fill])
    inA = dst_u < SPLIT
    def per_core_rows(flat, pad_val, c0, c1, ce):
        l0 = 16 * c0 * ce
        rows0 = flat[:l0].reshape(16, c0 * ce)
        rows1 = flat[l0:].reshape(16, c1 * ce)
        cm = max(c0, c1) * ce
        rows0 = jnp.pad(rows0, ((0, 0), (0, cm - c0 * ce)),
                        constant_values=pad_val)
        rows1 = jnp.pad(rows1, ((0, 0), (0, cm - c1 * ce)),
                        constant_values=pad_val)
        return jnp.stack([rows0, rows1], axis=1).reshape(NW, cm)

    src_r = per_core_rows(src_u, NP_ - 1, _C0, _C1, AGG_C)
    dstA_r = per_core_rows(jnp.where(inA, dst_u, SPLIT), SPLIT,
                           _C0, _C1, AGG_C)
    dstB_r = per_core_rows(jnp.where(inA, SPLIT, dst_u - SPLIT), SPLIT,
                           _C0, _C1, AGG_C)
    efill = jnp.zeros((E_PAD - E,), jnp.int32)
    row_r = per_core_rows(jnp.concatenate([row, efill]), 0,
                          _Q0, _Q1, CHUNK).reshape(NW, _QMAX, CHUNK)
    col_r = per_core_rows(jnp.concatenate([col, efill]), 0,
                          _Q0, _Q1, CHUNK).reshape(NW, _QMAX, CHUNK)

    zerosA = jnp.zeros((ACC_R, D), f32)
    zeros1d = jnp.zeros((NP_,), f32)
    ones1d = jnp.ones((CHUNK,), f32)
    x_pad = jnp.concatenate([x, jnp.zeros((NP_ - N, D), f32)], axis=0)

    degp = _deg_kernel(dst_r := dst_u.reshape(NW, A_CH, CHUNK), ones1d,
                       zeros1d)
    degp_t = degp.reshape(NC, NP_).T

    grid_n = NP_ // BLK
    z1, dinv = pl.pallas_call(
        _k1_body,
        grid=(grid_n,),
        in_specs=[
            pl.BlockSpec((BLK, D), _row_spec),
            pl.BlockSpec((D, D), _rep_spec),
            pl.BlockSpec((BLK, NC), _row_spec),
        ],
        out_specs=[
            pl.BlockSpec((BLK, D), _row_spec),
            pl.BlockSpec((BLK, 1), _row_spec),
        ],
        out_shape=[
            jax.ShapeDtypeStruct((NP_, D), f32),
            jax.ShapeDtypeStruct((NP_, 1), f32),
        ],
    )(x_pad, W1, degp_t)

    agg1 = _aggregate(z1, src_r, dstA_r, dstB_r, zerosA)

    z2 = pl.pallas_call(
        _k2_body,
        grid=(grid_n,),
        in_specs=[
            pl.BlockSpec((NC, BLK, D), _node_spec),
            pl.BlockSpec((BLK, D), _row_spec),
            pl.BlockSpec((BLK, 1), _row_spec),
            pl.BlockSpec((1, D), _rep_spec),
            pl.BlockSpec((D, D), _rep_spec),
        ],
        out_specs=pl.BlockSpec((BLK, D), _row_spec),
        out_shape=jax.ShapeDtypeStruct((NP_, D), f32),
    )(agg1, z1, dinv, b1.reshape(1, D), W2)

    agg2 = _aggregate(z2, src_r, dstA_r, dstB_r, zerosA)

    p, pneg = pl.pallas_call(
        _k3_body,
        grid=(grid_n,),
        in_specs=[
            pl.BlockSpec((NC, BLK, D), _node_spec),
            pl.BlockSpec((BLK, D), _row_spec),
            pl.BlockSpec((BLK, 1), _row_spec),
            pl.BlockSpec((1, D), _rep_spec),
            pl.BlockSpec((D, D), _rep_spec),
        ],
        out_specs=[
            pl.BlockSpec((BLK, D), _row_spec),
            pl.BlockSpec((BLK, D), _row_spec),
        ],
        out_shape=[
            jax.ShapeDtypeStruct((NP_, D), f32),
            jax.ShapeDtypeStruct((NP_, D), f32),
        ],
    )(agg2, z2, dinv, b2.reshape(1, D), M1)

    q = _q_kernel(p, pneg, row_r, col_r)

    m3p = jnp.concatenate([M3, jnp.zeros((D, 7), f32)], axis=1)
    c3t = jnp.broadcast_to(c3.reshape(1, 1), (1, 8))
    out = pl.pallas_call(
        _k4_body,
        grid=(E_PAD // EBLK,),
        in_specs=[
            pl.BlockSpec((EBLK, D), _row_spec),
            pl.BlockSpec((1, D), _rep_spec),
            pl.BlockSpec((D, D), _rep_spec),
            pl.BlockSpec((1, D), _rep_spec),
            pl.BlockSpec((D, 8), _rep_spec),
            pl.BlockSpec((1, 8), _rep_spec),
        ],
        out_specs=pl.BlockSpec((EBLK, 8), _row_spec),
        out_shape=jax.ShapeDtypeStruct((E_PAD, 8), f32),
    )(q, c1.reshape(1, D), M2, c2.reshape(1, D), m3p, c3t)

    return out[:E, :1]
